# baseline (speedup 1.0000x reference)
import functools

import jax
import jax.numpy as jnp
from jax import lax
from jax.experimental import pallas as pl
from jax.experimental.pallas import tpu as pltpu
from jax.experimental.pallas import tpu_sc as plsc

N = 10000
E = 160000
NW = 32
CHUNK = 128
NCH = 40
EPW = NCH * CHUNK
EP = NW * EPW
NPAD = 10240
RPT = NPAD // 16

_GD = lax.GatherDimensionNumbers(
    offset_dims=(), collapsed_slice_dims=(0,), start_index_map=(0,))


def _bcast_lane(v, l):
    il = jnp.full((16, 1), l, jnp.int32)
    return lax.gather(v, il, _GD, slice_sizes=(1,),
                      mode=lax.GatherScatterMode.PROMISE_IN_BOUNDS)


NBUF = 4


def _edge_pass(with_cnt):
    outs = [jax.ShapeDtypeStruct((2, NPAD, 16), jnp.float32)]
    if with_cnt:
        outs.append(jax.ShapeDtypeStruct((2, NPAD, 16), jnp.float32))
    scratch = [
        pltpu.VMEM((NCH, CHUNK), jnp.int32),
        pltpu.VMEM((NCH, CHUNK), jnp.int32),
        pltpu.VMEM((NCH, CHUNK), jnp.float32),
        pltpu.VMEM((NCH, CHUNK), jnp.float32),
        pltpu.VMEM((RPT, 16), jnp.float32),
        pltpu.VMEM_SHARED((NPAD, 16), jnp.float32),
    ]
    for _ in range(NBUF):
        scratch.append(pltpu.VMEM((CHUNK, 48), jnp.float32))
    for _ in range(NBUF):
        scratch.append(pltpu.VMEM((CHUNK, 16), jnp.float32))
    for _ in range(2 * NBUF):
        scratch.append(pltpu.SemaphoreType.DMA)
    if with_cnt:
        scratch.append(pltpu.VMEM((CHUNK, 16), jnp.float32))
        scratch.append(pltpu.VMEM_SHARED((NPAD, 16), jnp.float32))

    @functools.partial(
        pl.kernel,
        out_type=tuple(outs) if with_cnt else outs[0],
        mesh=plsc.VectorSubcoreMesh(core_axis_name="c", subcore_axis_name="s"),
        scratch_types=scratch,
        compiler_params=pltpu.CompilerParams(use_tc_tiling_on_sc=False),
    )
    def k(*refs):
        if with_cnt:
            (u_hbm, src_hbm, dst_hbm, ea0_hbm, ea1_hbm, out_hbm, cnt_hbm,
             src_v, dst_v, ea0_v, ea1_v, zbuf, acc, *rest) = refs
            rows = rest[0:NBUF]
            msg = rest[NBUF:2 * NBUF]
            gsem = rest[2 * NBUF:3 * NBUF]
            ssem = rest[3 * NBUF:4 * NBUF]
            ones_v, cacc = rest[4 * NBUF], rest[4 * NBUF + 1]
        else:
            (u_hbm, src_hbm, dst_hbm, ea0_hbm, ea1_hbm, out_hbm,
             src_v, dst_v, ea0_v, ea1_v, zbuf, acc, *rest) = refs
            rows = rest[0:NBUF]
            msg = rest[NBUF:2 * NBUF]
            gsem = rest[2 * NBUF:3 * NBUF]
            ssem = rest[3 * NBUF:4 * NBUF]
        cid = lax.axis_index("c")
        sid = lax.axis_index("s")
        w = cid * 16 + sid
        r0 = w * NCH

        pltpu.sync_copy(src_hbm.at[pl.ds(r0, NCH)], src_v)
        pltpu.sync_copy(dst_hbm.at[pl.ds(r0, NCH)], dst_v)
        pltpu.sync_copy(ea0_hbm.at[pl.ds(r0, NCH)], ea0_v)
        pltpu.sync_copy(ea1_hbm.at[pl.ds(r0, NCH)], ea1_v)

        z = jnp.zeros((16,), jnp.float32)
        one = jnp.ones((16,), jnp.float32)

        @plsc.parallel_loop(0, RPT, unroll=4)
        def zrow(r):
            zbuf[r, :] = z
        pltpu.sync_copy(zbuf, acc.at[pl.ds(sid * RPT, RPT)])
        if with_cnt:
            pltpu.sync_copy(zbuf, cacc.at[pl.ds(sid * RPT, RPT)])

            @plsc.parallel_loop(0, CHUNK, unroll=4)
            def orow(r):
                ones_v[r, :] = one
        plsc.subcore_barrier()

        def quad(q, _):
            c0 = q * NBUF
            gh = [pltpu.async_copy(u_hbm.at[src_v.at[c0 + b]], rows[b],
                                   gsem[b])
                  for b in range(NBUF)]
            sh = []
            for b in range(NBUF):
                c = c0 + b
                with jax.named_scope("gw"):
                    gh[b].wait()

                with jax.named_scope("cb"):
                    @plsc.parallel_loop(0, CHUNK // 16, unroll=2)
                    def grp(g, b=b, c=c):
                        base = g * 16
                        ea0g = ea0_v[c, pl.ds(base, 16)]
                        ea1g = ea1_v[c, pl.ds(base, 16)]
                        for l in range(16):
                            b0 = _bcast_lane(ea0g, l)
                            b1 = _bcast_lane(ea1g, l)
                            e = base + l
                            a = rows[b][e, pl.ds(0, 16)]
                            bb = rows[b][e, pl.ds(16, 16)]
                            cc = rows[b][e, pl.ds(32, 16)]
                            msg[b][e, :] = b0 * a + b1 * bb + cc
                with jax.named_scope("sc"):
                    sh.append(pltpu.async_copy(msg[b], acc.at[dst_v.at[c]],
                                               ssem[b], add=True))
                    if with_cnt:
                        pltpu.sync_copy(ones_v, cacc.at[dst_v.at[c]],
                                        add=True)
            with jax.named_scope("sd"):
                for h in sh:
                    h.wait()
            return 0
        lax.fori_loop(0, NCH // NBUF, quad, 0)

        plsc.subcore_barrier()
        pltpu.sync_copy(acc.at[pl.ds(sid * RPT, RPT)],
                        out_hbm.at[cid, pl.ds(sid * RPT, RPT)])
        if with_cnt:
            pltpu.sync_copy(cacc.at[pl.ds(sid * RPT, RPT)],
                            cnt_hbm.at[cid, pl.ds(sid * RPT, RPT)])

    return k


_EDGE_CNT = _edge_pass(True)
_EDGE = _edge_pass(False)


def _leaky(v):
    return jnp.where(v >= 0, v, 0.01 * v)


def _cat3(nnW, nnb, ic, oc):
    return jnp.concatenate(
        [nnW[0].reshape(ic, oc), nnW[1].reshape(ic, oc), nnb.reshape(ic, oc)],
        axis=1)


def kernel(x, edge_attr, nn1_W, nn1_b, root1, bias1, nn2_W, nn2_b, root2,
           bias2, nn3_W, nn3_b, root3, bias3, edge_index):
    padE = EP - E
    src = jnp.concatenate([edge_index[0], jnp.zeros((padE,), jnp.int32)])
    pad_dst = N + jnp.arange(padE, dtype=jnp.int32) % (NPAD - N)
    dst = jnp.concatenate([edge_index[1], pad_dst])
    ea0 = jnp.concatenate([edge_attr[:, 0], jnp.zeros((padE,), jnp.float32)])
    ea1 = jnp.concatenate([edge_attr[:, 1], jnp.zeros((padE,), jnp.float32)])
    src = src.reshape(NW * NCH, CHUNK)
    dst = dst.reshape(NW * NCH, CHUNK)
    ea0 = ea0.reshape(NW * NCH, CHUNK)
    ea1 = ea1.reshape(NW * NCH, CHUNK)

    M1 = _cat3(nn1_W, nn1_b, 2, 16)
    M2 = _cat3(nn2_W, nn2_b, 16, 16)
    M3 = jnp.zeros((64, 48), jnp.float32)
    M3 = M3.at[:, 0:2].set(nn3_W[0].reshape(64, 2))
    M3 = M3.at[:, 16:18].set(nn3_W[1].reshape(64, 2))
    M3 = M3.at[:, 32:34].set(nn3_b.reshape(64, 2))

    U1 = x @ M1
    agg1p, cntp = _EDGE_CNT(U1, src, dst, ea0, ea1)
    cnt = (cntp[0] + cntp[1])[:N, 0]
    inv = 1.0 / jnp.clip(cnt, 1.0, None)
    x1 = _leaky((agg1p[0] + agg1p[1])[:N] * inv[:, None] + x @ root1 + bias1)

    agg2p = _EDGE(x1 @ M2, src, dst, ea0, ea1)
    x2 = _leaky((agg2p[0] + agg2p[1])[:N] * inv[:, None] + x1 @ root2 + bias2)
    agg3p = _EDGE(x2 @ M2, src, dst, ea0, ea1)
    x3 = _leaky((agg3p[0] + agg3p[1])[:N] * inv[:, None] + x2 @ root2 + bias2)
    agg4p = _EDGE(x3 @ M2, src, dst, ea0, ea1)
    x4 = _leaky((agg4p[0] + agg4p[1])[:N] * inv[:, None] + x3 @ root2 + bias2)

    x5 = jnp.concatenate([x1, x2, x3, x4], axis=1)
    agg5p = _EDGE(x5 @ M3, src, dst, ea0, ea1)
    return (agg5p[0] + agg5p[1])[:N, 0:2] * inv[:, None] + x5 @ root3 + bias3


# speedup vs baseline: 1.7023x; 1.7023x over previous
import functools

import jax
import jax.numpy as jnp
from jax import lax
from jax.experimental import pallas as pl
from jax.experimental.pallas import tpu as pltpu
from jax.experimental.pallas import tpu_sc as plsc

N = 10000
E = 160000
NW = 32
CHUNK = 128
NCH = 40
EPW = NCH * CHUNK
EP = NW * EPW
NPAD = 10240
RPT = NPAD // 16

_GD = lax.GatherDimensionNumbers(
    offset_dims=(), collapsed_slice_dims=(0,), start_index_map=(0,))


def _bcast_lane(v, l):
    il = jnp.full((16, 1), l, jnp.int32)
    return lax.gather(v, il, _GD, slice_sizes=(1,),
                      mode=lax.GatherScatterMode.PROMISE_IN_BOUNDS)


NBUF = 4


def _edge_pass(with_cnt):
    outs = [jax.ShapeDtypeStruct((2, NPAD, 16), jnp.float32)]
    if with_cnt:
        outs.append(jax.ShapeDtypeStruct((2, NPAD, 16), jnp.float32))
    scratch = [
        pltpu.VMEM((NCH, CHUNK), jnp.int32),
        pltpu.VMEM((NCH, CHUNK), jnp.int32),
        pltpu.VMEM((NCH, CHUNK), jnp.float32),
        pltpu.VMEM((NCH, CHUNK), jnp.float32),
        pltpu.VMEM((RPT, 16), jnp.float32),
        pltpu.VMEM_SHARED((NPAD, 16), jnp.float32),
    ]
    for _ in range(NBUF):
        scratch.append(pltpu.VMEM((CHUNK, 48), jnp.float32))
    for _ in range(NBUF):
        scratch.append(pltpu.VMEM((CHUNK, 16), jnp.float32))
    for _ in range(2 * NBUF):
        scratch.append(pltpu.SemaphoreType.DMA)
    if with_cnt:
        scratch.append(pltpu.VMEM((CHUNK, 16), jnp.float32))
        scratch.append(pltpu.VMEM_SHARED((NPAD, 16), jnp.float32))

    @functools.partial(
        pl.kernel,
        out_type=tuple(outs) if with_cnt else outs[0],
        mesh=plsc.VectorSubcoreMesh(core_axis_name="c", subcore_axis_name="s"),
        scratch_types=scratch,
        compiler_params=pltpu.CompilerParams(use_tc_tiling_on_sc=False),
    )
    def k(*refs):
        if with_cnt:
            (u_hbm, src_hbm, dst_hbm, ea0_hbm, ea1_hbm, out_hbm, cnt_hbm,
             src_v, dst_v, ea0_v, ea1_v, zbuf, acc, *rest) = refs
            rows = rest[0:NBUF]
            msg = rest[NBUF:2 * NBUF]
            gsem = rest[2 * NBUF:3 * NBUF]
            ssem = rest[3 * NBUF:4 * NBUF]
            ones_v, cacc = rest[4 * NBUF], rest[4 * NBUF + 1]
        else:
            (u_hbm, src_hbm, dst_hbm, ea0_hbm, ea1_hbm, out_hbm,
             src_v, dst_v, ea0_v, ea1_v, zbuf, acc, *rest) = refs
            rows = rest[0:NBUF]
            msg = rest[NBUF:2 * NBUF]
            gsem = rest[2 * NBUF:3 * NBUF]
            ssem = rest[3 * NBUF:4 * NBUF]
        cid = lax.axis_index("c")
        sid = lax.axis_index("s")
        w = cid * 16 + sid
        r0 = w * NCH

        pltpu.sync_copy(src_hbm.at[pl.ds(r0, NCH)], src_v)
        pltpu.sync_copy(dst_hbm.at[pl.ds(r0, NCH)], dst_v)
        pltpu.sync_copy(ea0_hbm.at[pl.ds(r0, NCH)], ea0_v)
        pltpu.sync_copy(ea1_hbm.at[pl.ds(r0, NCH)], ea1_v)

        z = jnp.zeros((16,), jnp.float32)
        one = jnp.ones((16,), jnp.float32)

        @plsc.parallel_loop(0, RPT, unroll=4)
        def zrow(r):
            zbuf[r, :] = z
        pltpu.sync_copy(zbuf, acc.at[pl.ds(sid * RPT, RPT)])
        if with_cnt:
            pltpu.sync_copy(zbuf, cacc.at[pl.ds(sid * RPT, RPT)])

            @plsc.parallel_loop(0, CHUNK, unroll=4)
            def orow(r):
                ones_v[r, :] = one
        plsc.subcore_barrier()

        def quad(q, _):
            c0 = q * NBUF
            gh = [pltpu.async_copy(u_hbm.at[src_v.at[c0 + b]], rows[b],
                                   gsem[b])
                  for b in range(NBUF)]
            sh = []
            for b in range(NBUF):
                c = c0 + b
                with jax.named_scope("gw"):
                    gh[b].wait()

                with jax.named_scope("cb"):
                    @plsc.parallel_loop(0, CHUNK // 16, unroll=2)
                    def grp(g, b=b, c=c):
                        base = g * 16
                        ea0g = ea0_v[c, pl.ds(base, 16)]
                        ea1g = ea1_v[c, pl.ds(base, 16)]
                        for l in range(16):
                            b0 = _bcast_lane(ea0g, l)
                            b1 = _bcast_lane(ea1g, l)
                            e = base + l
                            a = rows[b][e, pl.ds(0, 16)]
                            bb = rows[b][e, pl.ds(16, 16)]
                            cc = rows[b][e, pl.ds(32, 16)]
                            msg[b][e, :] = b0 * a + b1 * bb + cc
                with jax.named_scope("sc"):
                    sh.append(pltpu.async_copy(msg[b], acc.at[dst_v.at[c]],
                                               ssem[b], add=True))
                    if with_cnt:
                        pltpu.sync_copy(ones_v, cacc.at[dst_v.at[c]],
                                        add=True)
            with jax.named_scope("sd"):
                for h in sh:
                    h.wait()
            return 0
        lax.fori_loop(0, NCH // NBUF, quad, 0)

        plsc.subcore_barrier()
        pltpu.sync_copy(acc.at[pl.ds(sid * RPT, RPT)],
                        out_hbm.at[cid, pl.ds(sid * RPT, RPT)])
        if with_cnt:
            pltpu.sync_copy(cacc.at[pl.ds(sid * RPT, RPT)],
                            cnt_hbm.at[cid, pl.ds(sid * RPT, RPT)])

    return k


_EDGE_CNT = _edge_pass(True)
_EDGE = _edge_pass(False)


def _leaky(v):
    return jnp.where(v >= 0, v, 0.01 * v)


def _cat3(nnW, nnb, ic, oc):
    return jnp.concatenate(
        [nnW[0].reshape(ic, oc), nnW[1].reshape(ic, oc), nnb.reshape(ic, oc)],
        axis=1)


def kernel(x, edge_attr, nn1_W, nn1_b, root1, bias1, nn2_W, nn2_b, root2,
           bias2, nn3_W, nn3_b, root3, bias3, edge_index):
    padE = EP - E
    pad_src = jnp.arange(padE, dtype=jnp.int32) % N
    pad_dst = N + jnp.arange(padE, dtype=jnp.int32) % (NPAD - N)
    src = jnp.concatenate([edge_index[0], pad_src])
    dst = jnp.concatenate([edge_index[1], pad_dst])
    ea0 = jnp.concatenate([edge_attr[:, 0], jnp.zeros((padE,), jnp.float32)])
    ea1 = jnp.concatenate([edge_attr[:, 1], jnp.zeros((padE,), jnp.float32)])

    def deal(a):
        return (a.reshape(NCH, NW, CHUNK).transpose(1, 0, 2)
                .reshape(NW * NCH, CHUNK))
    src = deal(src)
    dst = deal(dst)
    ea0 = deal(ea0)
    ea1 = deal(ea1)

    M1 = _cat3(nn1_W, nn1_b, 2, 16)
    M2 = _cat3(nn2_W, nn2_b, 16, 16)
    M3 = jnp.zeros((64, 48), jnp.float32)
    M3 = M3.at[:, 0:2].set(nn3_W[0].reshape(64, 2))
    M3 = M3.at[:, 16:18].set(nn3_W[1].reshape(64, 2))
    M3 = M3.at[:, 32:34].set(nn3_b.reshape(64, 2))

    U1 = x @ M1
    agg1p, cntp = _EDGE_CNT(U1, src, dst, ea0, ea1)
    cnt = (cntp[0] + cntp[1])[:N, 0]
    inv = 1.0 / jnp.clip(cnt, 1.0, None)
    x1 = _leaky((agg1p[0] + agg1p[1])[:N] * inv[:, None] + x @ root1 + bias1)

    agg2p = _EDGE(x1 @ M2, src, dst, ea0, ea1)
    x2 = _leaky((agg2p[0] + agg2p[1])[:N] * inv[:, None] + x1 @ root2 + bias2)
    agg3p = _EDGE(x2 @ M2, src, dst, ea0, ea1)
    x3 = _leaky((agg3p[0] + agg3p[1])[:N] * inv[:, None] + x2 @ root2 + bias2)
    agg4p = _EDGE(x3 @ M2, src, dst, ea0, ea1)
    x4 = _leaky((agg4p[0] + agg4p[1])[:N] * inv[:, None] + x3 @ root2 + bias2)

    x5 = jnp.concatenate([x1, x2, x3, x4], axis=1)
    agg5p = _EDGE(x5 @ M3, src, dst, ea0, ea1)
    return (agg5p[0] + agg5p[1])[:N, 0:2] * inv[:, None] + x5 @ root3 + bias3


# speedup vs baseline: 2.1100x; 1.2395x over previous
import functools

import jax
import jax.numpy as jnp
from jax import lax
from jax.experimental import pallas as pl
from jax.experimental.pallas import tpu as pltpu
from jax.experimental.pallas import tpu_sc as plsc

N = 10000
E = 160000
NW = 32
CHUNK = 128
NCH = 40
EPW = NCH * CHUNK
EP = NW * EPW
NPAD = 10240
RPT = NPAD // 16

_GD = lax.GatherDimensionNumbers(
    offset_dims=(), collapsed_slice_dims=(0,), start_index_map=(0,))


def _bcast_lane(v, l):
    il = jnp.full((16, 1), l, jnp.int32)
    return lax.gather(v, il, _GD, slice_sizes=(1,),
                      mode=lax.GatherScatterMode.PROMISE_IN_BOUNDS)


NBUF = 8


def _edge_pass(with_cnt):
    outs = [jax.ShapeDtypeStruct((2, NPAD, 16), jnp.float32)]
    if with_cnt:
        outs.append(jax.ShapeDtypeStruct((2, NPAD, 16), jnp.float32))
    scratch = [
        pltpu.VMEM((NCH, CHUNK), jnp.int32),
        pltpu.VMEM((NCH, CHUNK), jnp.int32),
        pltpu.VMEM((NCH, CHUNK), jnp.float32),
        pltpu.VMEM((NCH, CHUNK), jnp.float32),
        pltpu.VMEM((RPT, 16), jnp.float32),
        pltpu.VMEM_SHARED((NPAD, 16), jnp.float32),
    ]
    for _ in range(NBUF):
        scratch.append(pltpu.VMEM((CHUNK, 48), jnp.float32))
    for _ in range(NBUF):
        scratch.append(pltpu.VMEM((CHUNK, 16), jnp.float32))
    for _ in range(2 * NBUF):
        scratch.append(pltpu.SemaphoreType.DMA)
    if with_cnt:
        scratch.append(pltpu.VMEM((CHUNK, 16), jnp.float32))
        scratch.append(pltpu.VMEM_SHARED((NPAD, 16), jnp.float32))

    @functools.partial(
        pl.kernel,
        out_type=tuple(outs) if with_cnt else outs[0],
        mesh=plsc.VectorSubcoreMesh(core_axis_name="c", subcore_axis_name="s"),
        scratch_types=scratch,
        compiler_params=pltpu.CompilerParams(use_tc_tiling_on_sc=False),
    )
    def k(*refs):
        if with_cnt:
            (u_hbm, src_hbm, dst_hbm, ea0_hbm, ea1_hbm, out_hbm, cnt_hbm,
             src_v, dst_v, ea0_v, ea1_v, zbuf, acc, *rest) = refs
            rows = rest[0:NBUF]
            msg = rest[NBUF:2 * NBUF]
            gsem = rest[2 * NBUF:3 * NBUF]
            ssem = rest[3 * NBUF:4 * NBUF]
            ones_v, cacc = rest[4 * NBUF], rest[4 * NBUF + 1]
        else:
            (u_hbm, src_hbm, dst_hbm, ea0_hbm, ea1_hbm, out_hbm,
             src_v, dst_v, ea0_v, ea1_v, zbuf, acc, *rest) = refs
            rows = rest[0:NBUF]
            msg = rest[NBUF:2 * NBUF]
            gsem = rest[2 * NBUF:3 * NBUF]
            ssem = rest[3 * NBUF:4 * NBUF]
        cid = lax.axis_index("c")
        sid = lax.axis_index("s")
        w = cid * 16 + sid
        r0 = w * NCH

        pltpu.sync_copy(src_hbm.at[pl.ds(r0, NCH)], src_v)
        pltpu.sync_copy(dst_hbm.at[pl.ds(r0, NCH)], dst_v)
        pltpu.sync_copy(ea0_hbm.at[pl.ds(r0, NCH)], ea0_v)
        pltpu.sync_copy(ea1_hbm.at[pl.ds(r0, NCH)], ea1_v)

        z = jnp.zeros((16,), jnp.float32)
        one = jnp.ones((16,), jnp.float32)

        @plsc.parallel_loop(0, RPT, unroll=4)
        def zrow(r):
            zbuf[r, :] = z
        pltpu.sync_copy(zbuf, acc.at[pl.ds(sid * RPT, RPT)])
        if with_cnt:
            pltpu.sync_copy(zbuf, cacc.at[pl.ds(sid * RPT, RPT)])

            @plsc.parallel_loop(0, CHUNK, unroll=4)
            def orow(r):
                ones_v[r, :] = one
        plsc.subcore_barrier()

        def quad(q, _):
            c0 = q * NBUF
            gh = [pltpu.async_copy(u_hbm.at[src_v.at[c0 + b]], rows[b],
                                   gsem[b])
                  for b in range(NBUF)]
            sh = []
            for b in range(NBUF):
                c = c0 + b
                with jax.named_scope("gw"):
                    gh[b].wait()

                with jax.named_scope("cb"):
                    @plsc.parallel_loop(0, CHUNK // 16, unroll=4)
                    def grp(g, b=b, c=c):
                        base = g * 16
                        ea0g = ea0_v[c, pl.ds(base, 16)]
                        ea1g = ea1_v[c, pl.ds(base, 16)]
                        for l in range(16):
                            b0 = _bcast_lane(ea0g, l)
                            b1 = _bcast_lane(ea1g, l)
                            e = base + l
                            a = rows[b][e, pl.ds(0, 16)]
                            bb = rows[b][e, pl.ds(16, 16)]
                            cc = rows[b][e, pl.ds(32, 16)]
                            msg[b][e, :] = b0 * a + b1 * bb + cc
                with jax.named_scope("sc"):
                    sh.append(pltpu.async_copy(msg[b], acc.at[dst_v.at[c]],
                                               ssem[b], add=True))
                    if with_cnt:
                        pltpu.sync_copy(ones_v, cacc.at[dst_v.at[c]],
                                        add=True)
            with jax.named_scope("sd"):
                for h in sh:
                    h.wait()
            return 0
        lax.fori_loop(0, NCH // NBUF, quad, 0)

        plsc.subcore_barrier()
        pltpu.sync_copy(acc.at[pl.ds(sid * RPT, RPT)],
                        out_hbm.at[cid, pl.ds(sid * RPT, RPT)])
        if with_cnt:
            pltpu.sync_copy(cacc.at[pl.ds(sid * RPT, RPT)],
                            cnt_hbm.at[cid, pl.ds(sid * RPT, RPT)])

    return k


_EDGE_CNT = _edge_pass(True)
_EDGE = _edge_pass(False)


def _leaky(v):
    return jnp.where(v >= 0, v, 0.01 * v)


def _cat3(nnW, nnb, ic, oc):
    return jnp.concatenate(
        [nnW[0].reshape(ic, oc), nnW[1].reshape(ic, oc), nnb.reshape(ic, oc)],
        axis=1)


def kernel(x, edge_attr, nn1_W, nn1_b, root1, bias1, nn2_W, nn2_b, root2,
           bias2, nn3_W, nn3_b, root3, bias3, edge_index):
    padE = EP - E
    pad_src = jnp.arange(padE, dtype=jnp.int32) % N
    pad_dst = N + jnp.arange(padE, dtype=jnp.int32) % (NPAD - N)
    src = jnp.concatenate([edge_index[0], pad_src])
    dst = jnp.concatenate([edge_index[1], pad_dst])
    ea0 = jnp.concatenate([edge_attr[:, 0], jnp.zeros((padE,), jnp.float32)])
    ea1 = jnp.concatenate([edge_attr[:, 1], jnp.zeros((padE,), jnp.float32)])

    def deal(a):
        return (a.reshape(NCH, NW, CHUNK).transpose(1, 0, 2)
                .reshape(NW * NCH, CHUNK))
    src = deal(src)
    dst = deal(dst)
    ea0 = deal(ea0)
    ea1 = deal(ea1)

    M1 = _cat3(nn1_W, nn1_b, 2, 16)
    M2 = _cat3(nn2_W, nn2_b, 16, 16)
    M3 = jnp.zeros((64, 48), jnp.float32)
    M3 = M3.at[:, 0:2].set(nn3_W[0].reshape(64, 2))
    M3 = M3.at[:, 16:18].set(nn3_W[1].reshape(64, 2))
    M3 = M3.at[:, 32:34].set(nn3_b.reshape(64, 2))

    U1 = x @ M1
    agg1p, cntp = _EDGE_CNT(U1, src, dst, ea0, ea1)
    cnt = (cntp[0] + cntp[1])[:N, 0]
    inv = 1.0 / jnp.clip(cnt, 1.0, None)
    x1 = _leaky((agg1p[0] + agg1p[1])[:N] * inv[:, None] + x @ root1 + bias1)

    agg2p = _EDGE(x1 @ M2, src, dst, ea0, ea1)
    x2 = _leaky((agg2p[0] + agg2p[1])[:N] * inv[:, None] + x1 @ root2 + bias2)
    agg3p = _EDGE(x2 @ M2, src, dst, ea0, ea1)
    x3 = _leaky((agg3p[0] + agg3p[1])[:N] * inv[:, None] + x2 @ root2 + bias2)
    agg4p = _EDGE(x3 @ M2, src, dst, ea0, ea1)
    x4 = _leaky((agg4p[0] + agg4p[1])[:N] * inv[:, None] + x3 @ root2 + bias2)

    x5 = jnp.concatenate([x1, x2, x3, x4], axis=1)
    agg5p = _EDGE(x5 @ M3, src, dst, ea0, ea1)
    return (agg5p[0] + agg5p[1])[:N, 0:2] * inv[:, None] + x5 @ root3 + bias3


# speedup vs baseline: 2.4501x; 1.1612x over previous
import functools

import jax
import jax.numpy as jnp
from jax import lax
from jax.experimental import pallas as pl
from jax.experimental.pallas import tpu as pltpu
from jax.experimental.pallas import tpu_sc as plsc

N = 10000
E = 160000
NW = 32
CHUNK = 128
NCH = 40
EPW = NCH * CHUNK
EP = NW * EPW
NPAD = 10240
RPT = NPAD // 16

_GD = lax.GatherDimensionNumbers(
    offset_dims=(), collapsed_slice_dims=(0,), start_index_map=(0,))


def _bcast_lane(v, l):
    il = jnp.full((16, 1), l, jnp.int32)
    return lax.gather(v, il, _GD, slice_sizes=(1,),
                      mode=lax.GatherScatterMode.PROMISE_IN_BOUNDS)


NBUF = 8


def _edge_pass(with_cnt):
    outs = [jax.ShapeDtypeStruct((2, NPAD, 16), jnp.float32)]
    if with_cnt:
        outs.append(jax.ShapeDtypeStruct((2, NPAD, 16), jnp.float32))
    scratch = [
        pltpu.VMEM((NCH, CHUNK), jnp.int32),
        pltpu.VMEM((NCH, CHUNK), jnp.int32),
        pltpu.VMEM((NCH, CHUNK), jnp.float32),
        pltpu.VMEM((NCH, CHUNK), jnp.float32),
        pltpu.VMEM((RPT, 16), jnp.float32),
        pltpu.VMEM_SHARED((NPAD, 16), jnp.float32),
    ]
    for _ in range(NBUF):
        scratch.append(pltpu.VMEM((CHUNK, 48), jnp.float32))
    for _ in range(NBUF):
        scratch.append(pltpu.VMEM((CHUNK, 16), jnp.float32))
    for _ in range(2 * NBUF):
        scratch.append(pltpu.SemaphoreType.DMA)
    if with_cnt:
        scratch.append(pltpu.VMEM((CHUNK, 16), jnp.float32))
        scratch.append(pltpu.VMEM_SHARED((NPAD, 16), jnp.float32))

    @functools.partial(
        pl.kernel,
        out_type=tuple(outs) if with_cnt else outs[0],
        mesh=plsc.VectorSubcoreMesh(core_axis_name="c", subcore_axis_name="s"),
        scratch_types=scratch,
        compiler_params=pltpu.CompilerParams(use_tc_tiling_on_sc=False),
    )
    def k(*refs):
        if with_cnt:
            (u_hbm, src_hbm, dst_hbm, ea0_hbm, ea1_hbm, out_hbm, cnt_hbm,
             src_v, dst_v, ea0_v, ea1_v, zbuf, acc, *rest) = refs
            rows = rest[0:NBUF]
            msg = rest[NBUF:2 * NBUF]
            gsem = rest[2 * NBUF:3 * NBUF]
            ssem = rest[3 * NBUF:4 * NBUF]
            ones_v, cacc = rest[4 * NBUF], rest[4 * NBUF + 1]
        else:
            (u_hbm, src_hbm, dst_hbm, ea0_hbm, ea1_hbm, out_hbm,
             src_v, dst_v, ea0_v, ea1_v, zbuf, acc, *rest) = refs
            rows = rest[0:NBUF]
            msg = rest[NBUF:2 * NBUF]
            gsem = rest[2 * NBUF:3 * NBUF]
            ssem = rest[3 * NBUF:4 * NBUF]
        cid = lax.axis_index("c")
        sid = lax.axis_index("s")
        w = cid * 16 + sid
        r0 = w * NCH

        st = [pltpu.async_copy(src_hbm.at[pl.ds(r0, NCH)], src_v, gsem[0]),
              pltpu.async_copy(dst_hbm.at[pl.ds(r0, NCH)], dst_v, gsem[1]),
              pltpu.async_copy(ea0_hbm.at[pl.ds(r0, NCH)], ea0_v, gsem[2]),
              pltpu.async_copy(ea1_hbm.at[pl.ds(r0, NCH)], ea1_v, gsem[3])]

        z = jnp.zeros((16,), jnp.float32)
        one = jnp.ones((16,), jnp.float32)

        @plsc.parallel_loop(0, RPT, unroll=4)
        def zrow(r):
            zbuf[r, :] = z
        if with_cnt:
            @plsc.parallel_loop(0, CHUNK, unroll=4)
            def orow(r):
                ones_v[r, :] = one
        for h in st:
            h.wait()
        pltpu.sync_copy(zbuf, acc.at[pl.ds(sid * RPT, RPT)])
        if with_cnt:
            pltpu.sync_copy(zbuf, cacc.at[pl.ds(sid * RPT, RPT)])
        plsc.subcore_barrier()

        NOCT = NCH // NBUF
        for b in range(NBUF):
            pltpu.async_copy(u_hbm.at[src_v.at[b]], rows[b], gsem[b])

        def octave(q, _):
            c0 = q * NBUF
            sh = []
            for b in range(NBUF):
                c = c0 + b
                pltpu.make_async_copy(u_hbm.at[src_v.at[c]], rows[b],
                                      gsem[b]).wait()

                @plsc.parallel_loop(0, CHUNK // 16, unroll=4)
                def grp(g, b=b, c=c):
                    base = g * 16
                    ea0g = ea0_v[c, pl.ds(base, 16)]
                    ea1g = ea1_v[c, pl.ds(base, 16)]
                    for l in range(16):
                        b0 = _bcast_lane(ea0g, l)
                        b1 = _bcast_lane(ea1g, l)
                        e = base + l
                        a = rows[b][e, pl.ds(0, 16)]
                        bb = rows[b][e, pl.ds(16, 16)]
                        cc = rows[b][e, pl.ds(32, 16)]
                        msg[b][e, :] = b0 * a + b1 * bb + cc
                sh.append(pltpu.async_copy(msg[b], acc.at[dst_v.at[c]],
                                           ssem[b], add=True))
                if with_cnt:
                    pltpu.sync_copy(ones_v, cacc.at[dst_v.at[c]], add=True)

                @pl.when(q < NOCT - 1)
                def _refire(b=b, c=c):
                    pltpu.async_copy(u_hbm.at[src_v.at[c + NBUF]], rows[b],
                                     gsem[b])
            for h in sh:
                h.wait()
            return 0
        lax.fori_loop(0, NOCT, octave, 0)

        plsc.subcore_barrier()
        pltpu.sync_copy(acc.at[pl.ds(sid * RPT, RPT)],
                        out_hbm.at[cid, pl.ds(sid * RPT, RPT)])
        if with_cnt:
            pltpu.sync_copy(cacc.at[pl.ds(sid * RPT, RPT)],
                            cnt_hbm.at[cid, pl.ds(sid * RPT, RPT)])

    return k


_EDGE_CNT = _edge_pass(True)
_EDGE = _edge_pass(False)


def _leaky(v):
    return jnp.where(v >= 0, v, 0.01 * v)


def _cat3(nnW, nnb, ic, oc):
    return jnp.concatenate(
        [nnW[0].reshape(ic, oc), nnW[1].reshape(ic, oc), nnb.reshape(ic, oc)],
        axis=1)


def kernel(x, edge_attr, nn1_W, nn1_b, root1, bias1, nn2_W, nn2_b, root2,
           bias2, nn3_W, nn3_b, root3, bias3, edge_index):
    padE = EP - E
    pad_src = jnp.arange(padE, dtype=jnp.int32) % N
    pad_dst = N + jnp.arange(padE, dtype=jnp.int32) % (NPAD - N)
    src = jnp.concatenate([edge_index[0], pad_src])
    dst = jnp.concatenate([edge_index[1], pad_dst])
    ea0 = jnp.concatenate([edge_attr[:, 0], jnp.zeros((padE,), jnp.float32)])
    ea1 = jnp.concatenate([edge_attr[:, 1], jnp.zeros((padE,), jnp.float32)])

    def deal(a):
        return (a.reshape(NCH, NW, CHUNK).transpose(1, 0, 2)
                .reshape(NW * NCH, CHUNK))
    src = deal(src)
    dst = deal(dst)
    ea0 = deal(ea0)
    ea1 = deal(ea1)

    M1 = _cat3(nn1_W, nn1_b, 2, 16)
    M2 = _cat3(nn2_W, nn2_b, 16, 16)
    M3 = jnp.zeros((64, 48), jnp.float32)
    M3 = M3.at[:, 0:2].set(nn3_W[0].reshape(64, 2))
    M3 = M3.at[:, 16:18].set(nn3_W[1].reshape(64, 2))
    M3 = M3.at[:, 32:34].set(nn3_b.reshape(64, 2))

    U1 = x @ M1
    agg1p, cntp = _EDGE_CNT(U1, src, dst, ea0, ea1)
    cnt = (cntp[0] + cntp[1])[:N, 0]
    inv = 1.0 / jnp.clip(cnt, 1.0, None)
    x1 = _leaky((agg1p[0] + agg1p[1])[:N] * inv[:, None] + x @ root1 + bias1)

    agg2p = _EDGE(x1 @ M2, src, dst, ea0, ea1)
    x2 = _leaky((agg2p[0] + agg2p[1])[:N] * inv[:, None] + x1 @ root2 + bias2)
    agg3p = _EDGE(x2 @ M2, src, dst, ea0, ea1)
    x3 = _leaky((agg3p[0] + agg3p[1])[:N] * inv[:, None] + x2 @ root2 + bias2)
    agg4p = _EDGE(x3 @ M2, src, dst, ea0, ea1)
    x4 = _leaky((agg4p[0] + agg4p[1])[:N] * inv[:, None] + x3 @ root2 + bias2)

    x5 = jnp.concatenate([x1, x2, x3, x4], axis=1)
    agg5p = _EDGE(x5 @ M3, src, dst, ea0, ea1)
    return (agg5p[0] + agg5p[1])[:N, 0:2] * inv[:, None] + x5 @ root3 + bias3
